# baseline (speedup 1.0000x reference)
import functools

import jax
import jax.numpy as jnp
import numpy as np
from jax import lax
from jax.experimental import pallas as pl
from jax.experimental.pallas import tpu as pltpu
from jax.experimental.pallas import tpu_sc as plsc

N = 10000
H = 128
CW = 144
NR = 10240
K = 96
NW = 32
CH = 108
B = 6
NB = CH // B
EPW = K * CH
E_PAD = NW * EPW
IDX_ROWS = NW * CH + 4 * B
G = 64
BN = 200
GRID = N // BN
RPT = NR // 16
CC = 80


def _make_sc_edge(heads):
    mesh = plsc.VectorSubcoreMesh(core_axis_name="c", subcore_axis_name="s")

    @functools.partial(
        pl.kernel,
        out_type=jax.ShapeDtypeStruct((2, NR, CW), jnp.float32),
        mesh=mesh,
        scratch_types=[
            pltpu.VMEM((K, CW), jnp.float32),
            pltpu.VMEM((K, CW), jnp.float32),
            pltpu.VMEM((K, 16), jnp.float32),
            pltpu.VMEM((K, 16), jnp.float32),
            pltpu.VMEM((B, K), jnp.int32),
            pltpu.VMEM((B, K), jnp.int32),
            pltpu.VMEM((B, K), jnp.int32),
            pltpu.VMEM((B, K), jnp.int32),
            pltpu.VMEM_SHARED((NR, CW), jnp.float32),
        ] + [pltpu.SemaphoreType.DMA] * 8,
        compiler_params=pltpu.CompilerParams(use_tc_tiling_on_sc=False),
    )
    def sc_edge(ht, dt, src, dst, out, gb0, gb1, db0, db1, is0, is1, id0, id1,
                acc, g0h, g1h, g0d, g1d, b0s, b1s, b0d, b1d):
        c = lax.axis_index("c")
        s = lax.axis_index("s")
        w = s * 2 + c
        rbase = w * CH

        gbufs, dbufs = (gb0, gb1), (db0, db1)
        isrc, idst = (is0, is1), (id0, id1)
        ghs, gds = (g0h, g1h), (g0d, g1d)
        ibs, ibd = (b0s, b1s), (b0d, b1d)

        zero = jnp.zeros((16,), jnp.float32)

        def zrow(i, carry):
            for k in range(CW // 16):
                gb0[i, pl.ds(16 * k, 16)] = zero
            return carry

        lax.fori_loop(0, CC, zrow, 0)
        for t in range(RPT // CC):
            pltpu.sync_copy(gb0.at[pl.ds(0, CC)],
                            acc.at[pl.ds(s * RPT + t * CC, CC)])
        plsc.subcore_barrier()

        def blk_issue(bi, n):
            r = rbase + n * B
            pltpu.async_copy(src.at[pl.ds(r, B)], isrc[bi], ibs[bi])
            pltpu.async_copy(dst.at[pl.ds(r, B)], idst[bi], ibd[bi])

        def blk_drain(bi):
            pltpu.make_async_copy(src.at[pl.ds(0, B)], isrc[bi], ibs[bi]).wait()
            pltpu.make_async_copy(dst.at[pl.ds(0, B)], idst[bi], ibd[bi]).wait()

        def gather_issue(gi, bi, j):
            pltpu.async_copy(ht.at[isrc[bi].at[j]], gbufs[gi], ghs[gi])
            pltpu.async_copy(dt.at[idst[bi].at[j]], dbufs[gi], gds[gi])

        def gather_drain(gi):
            pltpu.make_async_copy(ht.at[isrc[0].at[0]], gbufs[gi],
                                  ghs[gi]).wait()
            pltpu.make_async_copy(dt.at[idst[0].at[0]], dbufs[gi],
                                  gds[gi]).wait()

        def compute_scatter(gi, bi, j):
            gb, db = gbufs[gi], dbufs[gi]

            def edge(i2, carry2):
                for u in range(2):
                    i = i2 * 2 + u
                    lg = gb[i, pl.ds(H, 16)] + db[i, :]
                    p = jnp.exp(jnp.where(lg >= 0.0, lg, lg * 0.2))
                    gb[i, pl.ds(H, 16)] = p
                    if heads == 1:
                        p0 = p.at[jnp.zeros((16,), jnp.int32)].get(
                            mode="promise_in_bounds")
                    for j8 in range(8):
                        pj = (p.at[jnp.full((16,), j8, jnp.int32)].get(
                                  mode="promise_in_bounds")
                              if heads == 8 else p0)
                        gb[i, pl.ds(16 * j8, 16)] = (
                            gb[i, pl.ds(16 * j8, 16)] * pj)
                return carry2

            lax.fori_loop(0, K // 2, edge, 0)
            pltpu.sync_copy(gb, acc.at[idst[bi].at[j]], add=True)

        pltpu.sync_copy(src.at[pl.ds(rbase, B)], is0)
        pltpu.sync_copy(dst.at[pl.ds(rbase, B)], id0)
        gather_issue(0, 0, 0)
        blk_issue(1, 1)

        def body(m, carry):
            for t in range(2 * B):
                gi = t % 2
                nt = t + 1
                if nt == B:
                    blk_drain(1)
                if nt == 2 * B:
                    blk_drain(0)
                gather_issue(1 - gi, (nt // B) % 2, nt % B)
                gather_drain(gi)
                compute_scatter(gi, t // B, t % B)
                if t == B - 1:
                    blk_issue(0, 2 * m + 2)
                if t == 2 * B - 1:
                    blk_issue(1, 2 * m + 3)
            return carry

        lax.fori_loop(0, NB // 2, body, 0)
        gather_drain(0)
        blk_drain(1)
        plsc.subcore_barrier()

        for t in range(RPT // CC):
            r = s * RPT + t * CC
            pltpu.sync_copy(acc.at[pl.ds(r, CC)], gb0.at[pl.ds(0, CC)])
            pltpu.sync_copy(gb0.at[pl.ds(0, CC)], out.at[c, pl.ds(r, CC)])

    return sc_edge


_sc8 = _make_sc_edge(8)
_sc1 = _make_sc_edge(1)


def _pre1_body(x_ref, W_ref, As_ref, Ad_ref, ht_ref, dt_ref):
    h = jnp.dot(x_ref[...], W_ref[...], preferred_element_type=jnp.float32)
    sa = jnp.dot(h, As_ref[...], preferred_element_type=jnp.float32)
    ht_ref[...] = jnp.concatenate([h, sa], axis=1)
    dt_ref[...] = jnp.dot(h, Ad_ref[...], preferred_element_type=jnp.float32)


_pre1 = pl.pallas_call(
    _pre1_body,
    grid=(GRID,),
    in_specs=[
        pl.BlockSpec((BN, H), lambda i: (i, 0)),
        pl.BlockSpec((H, H), lambda i: (0, 0)),
        pl.BlockSpec((H, 16), lambda i: (0, 0)),
        pl.BlockSpec((H, 16), lambda i: (0, 0)),
    ],
    out_specs=[
        pl.BlockSpec((BN, CW), lambda i: (i, 0)),
        pl.BlockSpec((BN, 16), lambda i: (i, 0)),
    ],
    out_shape=[
        jax.ShapeDtypeStruct((N, CW), jnp.float32),
        jax.ShapeDtypeStruct((N, 16), jnp.float32),
    ],
)


def _epilogue(num, R_ref, b_ref, g_ref, be_ref):
    nsum = num[0] + num[1]
    den = jnp.dot(nsum[:, H:], R_ref[...],
                  preferred_element_type=jnp.float32) + 1e-16
    gat = nsum[:, :H] / den + b_ref[...]
    xe = jnp.where(gat > 0, gat, jnp.exp(gat) - 1.0)
    mu = jnp.mean(xe, axis=1, keepdims=True)
    var = jnp.mean((xe - mu) ** 2, axis=1, keepdims=True)
    return (xe - mu) / jnp.sqrt(var + 1e-5) * g_ref[...] + be_ref[...]


def _make_mid(has_res):
    def body(*refs):
        if has_res:
            (num_ref, res_ref, R_ref, b_ref, g_ref, be_ref,
             W_ref, As_ref, Ad_ref, x_ref, ht_ref, dt_ref) = refs
        else:
            (num_ref, R_ref, b_ref, g_ref, be_ref,
             W_ref, As_ref, Ad_ref, x_ref, ht_ref, dt_ref) = refs
        xn = _epilogue(num_ref[...], R_ref, b_ref, g_ref, be_ref)
        if has_res:
            xn = xn + res_ref[...]
        x_ref[...] = xn
        h = jnp.dot(xn, W_ref[...], preferred_element_type=jnp.float32)
        sa = jnp.dot(h, As_ref[...], preferred_element_type=jnp.float32)
        ht_ref[...] = jnp.concatenate([h, sa], axis=1)
        dt_ref[...] = jnp.dot(h, Ad_ref[...], preferred_element_type=jnp.float32)

    in_specs = [pl.BlockSpec((2, BN, CW), lambda i: (0, i, 0))]
    if has_res:
        in_specs.append(pl.BlockSpec((BN, H), lambda i: (i, 0)))
    in_specs += [
        pl.BlockSpec((16, H), lambda i: (0, 0)),
        pl.BlockSpec((1, H), lambda i: (0, 0)),
        pl.BlockSpec((1, H), lambda i: (0, 0)),
        pl.BlockSpec((1, H), lambda i: (0, 0)),
        pl.BlockSpec((H, H), lambda i: (0, 0)),
        pl.BlockSpec((H, 16), lambda i: (0, 0)),
        pl.BlockSpec((H, 16), lambda i: (0, 0)),
    ]
    return pl.pallas_call(
        body,
        grid=(GRID,),
        in_specs=in_specs,
        out_specs=[
            pl.BlockSpec((BN, H), lambda i: (i, 0)),
            pl.BlockSpec((BN, CW), lambda i: (i, 0)),
            pl.BlockSpec((BN, 16), lambda i: (i, 0)),
        ],
        out_shape=[
            jax.ShapeDtypeStruct((N, H), jnp.float32),
            jax.ShapeDtypeStruct((N, CW), jnp.float32),
            jax.ShapeDtypeStruct((N, 16), jnp.float32),
        ],
    )


_mid_nores = _make_mid(False)
_mid_res = _make_mid(True)


def _post_body(num_ref, res_ref, batch_ref, R_ref, b_ref, g_ref, be_ref,
               Wl1_ref, bl1_ref, Wl2_ref, bl2_ref, o_ref, acc, cnt):
    i = pl.program_id(0)
    h3 = _epilogue(num_ref[...], R_ref, b_ref, g_ref, be_ref) + res_ref[...]
    bvec = batch_ref[0, 0, :]
    onehot = (bvec[:, None] ==
              lax.broadcasted_iota(jnp.int32, (BN, G), 1)).astype(jnp.float32)
    dn = (((0,), (0,)), ((), ()))
    contrib = lax.dot_general(onehot, h3, dn, preferred_element_type=jnp.float32)
    ccontrib = lax.dot_general(onehot, jnp.ones((BN, H), jnp.float32), dn,
                               preferred_element_type=jnp.float32)

    @pl.when(i == 0)
    def _():
        acc[...] = contrib
        cnt[...] = ccontrib

    @pl.when(i > 0)
    def _():
        acc[...] = acc[...] + contrib
        cnt[...] = cnt[...] + ccontrib

    @pl.when(i == GRID - 1)
    def _():
        pooled = acc[...] / jnp.maximum(cnt[...], 1.0)
        t = jnp.dot(pooled, Wl1_ref[...],
                    preferred_element_type=jnp.float32) + bl1_ref[...]
        t = jnp.where(t > 0, t, jnp.exp(t) - 1.0)
        o_ref[...] = jnp.dot(t, Wl2_ref[...],
                             preferred_element_type=jnp.float32) + bl2_ref[...]


_post = pl.pallas_call(
    _post_body,
    grid=(GRID,),
    in_specs=[
        pl.BlockSpec((2, BN, CW), lambda i: (0, i, 0)),
        pl.BlockSpec((BN, H), lambda i: (i, 0)),
        pl.BlockSpec((1, 1, BN), lambda i: (i, 0, 0)),
        pl.BlockSpec((16, H), lambda i: (0, 0)),
        pl.BlockSpec((1, H), lambda i: (0, 0)),
        pl.BlockSpec((1, H), lambda i: (0, 0)),
        pl.BlockSpec((1, H), lambda i: (0, 0)),
        pl.BlockSpec((H, H), lambda i: (0, 0)),
        pl.BlockSpec((1, H), lambda i: (0, 0)),
        pl.BlockSpec((H, H), lambda i: (0, 0)),
        pl.BlockSpec((1, H), lambda i: (0, 0)),
    ],
    out_specs=pl.BlockSpec((G, H), lambda i: (0, 0)),
    out_shape=jax.ShapeDtypeStruct((G, H), jnp.float32),
    scratch_shapes=[
        pltpu.VMEM((G, H), jnp.float32),
        pltpu.VMEM((G, H), jnp.float32),
    ],
)


def kernel(x, edge_index, batch, W1, as1, ad1, b1, g1, be1, W2, as2, ad2, b2,
           g2, be2, W3, as3, ad3, b3, g3, be3, Wl1, bl1, Wl2, bl2):
    f32 = jnp.float32
    E = edge_index.shape[1]
    pad = IDX_ROWS * K - N - E
    loops = jnp.arange(N, dtype=jnp.int32)
    src = jnp.concatenate(
        [edge_index[0].astype(jnp.int32), loops,
         jnp.zeros((pad,), jnp.int32)]).reshape(IDX_ROWS, K)
    dst = jnp.concatenate(
        [edge_index[1].astype(jnp.int32), loops,
         jnp.full((pad,), N, jnp.int32)]).reshape(IDX_ROWS, K)

    eye8 = jnp.eye(8, dtype=f32)

    def head_proj(a):
        m = (eye8[:, None, :] * a[:, :, None]).reshape(H, 8)
        return jnp.pad(m, ((0, 0), (0, 8)))

    def one_proj(a):
        return jnp.pad(a.T, ((0, 0), (0, 15)))

    As1, Ad1 = head_proj(as1), head_proj(ad1)
    As2, Ad2 = one_proj(as2), one_proj(ad2)
    As3, Ad3 = one_proj(as3), one_proj(ad3)

    R8 = np.zeros((16, H), np.float32)
    for hh in range(8):
        R8[hh, 16 * hh:16 * hh + 16] = 1.0
    R8 = jnp.asarray(R8)
    R1 = np.zeros((16, H), np.float32)
    R1[0, :] = 1.0
    R1 = jnp.asarray(R1)

    rb = lambda v: v.reshape(1, H)
    batch3 = batch.astype(jnp.int32).reshape(GRID, 1, BN)

    ht1, dt1 = _pre1(x, W1, As1, Ad1)
    num1 = _sc8(ht1, dt1, src, dst)
    h1, ht2, dt2 = _mid_nores(num1, R8, rb(b1), rb(g1), rb(be1), W2, As2, Ad2)
    num2 = _sc1(ht2, dt2, src, dst)
    h2, ht3, dt3 = _mid_res(num2, h1, R1, rb(b2), rb(g2), rb(be2), W3, As3, Ad3)
    num3 = _sc1(ht3, dt3, src, dst)
    return _post(num3, h2, batch3, R1, rb(b3), rb(g3), rb(be3),
                 Wl1, rb(bl1), Wl2, rb(bl2))


# speedup vs baseline: 1.1023x; 1.1023x over previous
import functools

import jax
import jax.numpy as jnp
import numpy as np
from jax import lax
from jax.experimental import pallas as pl
from jax.experimental.pallas import tpu as pltpu
from jax.experimental.pallas import tpu_sc as plsc

N = 10000
H = 128
CW = 144
NR = 10240
K = 72
NW = 32
CH = 144
B = 6
NB = CH // B
EPW = K * CH
E_PAD = NW * EPW
IDX_ROWS = NW * CH + 8
G = 64
BN = 200
GRID = N // BN
RPT = NR // 16
CC = 64


def _make_sc_edge(heads):
    mesh = plsc.VectorSubcoreMesh(core_axis_name="c", subcore_axis_name="s")

    @functools.partial(
        pl.kernel,
        out_type=jax.ShapeDtypeStruct((2, NR, CW), jnp.float32),
        mesh=mesh,
        scratch_types=[
            pltpu.VMEM((K, CW), jnp.float32),
            pltpu.VMEM((K, CW), jnp.float32),
            pltpu.VMEM((K, CW), jnp.float32),
            pltpu.VMEM((K, 16), jnp.float32),
            pltpu.VMEM((K, 16), jnp.float32),
            pltpu.VMEM((K, 16), jnp.float32),
            pltpu.VMEM((B, K), jnp.int32),
            pltpu.VMEM((B, K), jnp.int32),
            pltpu.VMEM((B, K), jnp.int32),
            pltpu.VMEM((B, K), jnp.int32),
            pltpu.VMEM_SHARED((NR, CW), jnp.float32),
        ] + [pltpu.SemaphoreType.DMA] * 13,
        compiler_params=pltpu.CompilerParams(use_tc_tiling_on_sc=False),
    )
    def sc_edge(ht, dt, src, dst, out, gb0, gb1, gb2, db0, db1, db2,
                is0, is1, id0, id1, acc,
                g0h, g1h, g2h, g0d, g1d, g2d, s0, s1, s2, b0s, b1s, b0d, b1d):
        c = lax.axis_index("c")
        s = lax.axis_index("s")
        w = s * 2 + c
        rbase = w * CH

        gbufs, dbufs = (gb0, gb1, gb2), (db0, db1, db2)
        isrc, idst = (is0, is1), (id0, id1)
        ghs, gds = (g0h, g1h, g2h), (g0d, g1d, g2d)
        scs = (s0, s1, s2)
        ibs, ibd = (b0s, b1s), (b0d, b1d)

        zero = jnp.zeros((16,), jnp.float32)

        def zrow(i, carry):
            for k in range(CW // 16):
                gb0[i, pl.ds(16 * k, 16)] = zero
            return carry

        lax.fori_loop(0, CC, zrow, 0)
        for t in range(RPT // CC):
            pltpu.sync_copy(gb0.at[pl.ds(0, CC)],
                            acc.at[pl.ds(s * RPT + t * CC, CC)])
        plsc.subcore_barrier()

        def blk_issue(bi, n):
            r = rbase + n * B
            pltpu.async_copy(src.at[pl.ds(r, B)], isrc[bi], ibs[bi])
            pltpu.async_copy(dst.at[pl.ds(r, B)], idst[bi], ibd[bi])

        def blk_drain(bi):
            pltpu.make_async_copy(src.at[pl.ds(0, B)], isrc[bi], ibs[bi]).wait()
            pltpu.make_async_copy(dst.at[pl.ds(0, B)], idst[bi], ibd[bi]).wait()

        def gather_issue(gi, bi, j):
            pltpu.async_copy(ht.at[isrc[bi].at[j]], gbufs[gi], ghs[gi])
            pltpu.async_copy(dt.at[idst[bi].at[j]], dbufs[gi], gds[gi])

        def gather_drain(gi):
            pltpu.make_async_copy(ht.at[isrc[0].at[0]], gbufs[gi],
                                  ghs[gi]).wait()
            pltpu.make_async_copy(dt.at[idst[0].at[0]], dbufs[gi],
                                  gds[gi]).wait()

        def compute(gi):
            gb, db = gbufs[gi], dbufs[gi]

            def edge(i2, carry2):
                for u in range(2):
                    i = i2 * 2 + u
                    lg = gb[i, pl.ds(H, 16)] + db[i, :]
                    p = jnp.exp(jnp.where(lg >= 0.0, lg, lg * 0.2))
                    gb[i, pl.ds(H, 16)] = p
                    if heads == 1:
                        p0 = p.at[jnp.zeros((16,), jnp.int32)].get(
                            mode="promise_in_bounds")
                    for j8 in range(8):
                        pj = (p.at[jnp.full((16,), j8, jnp.int32)].get(
                                  mode="promise_in_bounds")
                              if heads == 8 else p0)
                        gb[i, pl.ds(16 * j8, 16)] = (
                            gb[i, pl.ds(16 * j8, 16)] * pj)
                return carry2

            lax.fori_loop(0, K // 2, edge, 0)

        def scat_issue(gi, bi, j):
            pltpu.async_copy(gbufs[gi], acc.at[idst[bi].at[j]], scs[gi],
                             add=True)

        def scat_drain(gi):
            pltpu.make_async_copy(gbufs[gi], acc.at[idst[0].at[0]],
                                  scs[gi]).wait()

        def step(m, t, first):
            gi = t % 3
            gather_drain(gi)
            compute(gi)
            scat_issue(gi, t // B, t % B)
            if not first:
                scat_drain((t + 2) % 3)
            if t == 0:
                blk_issue(1, 2 * m + 1)
            if t == 4:
                blk_drain(1)
            if t == 6:
                blk_issue(0, 2 * m + 2)
            if t == 10:
                blk_drain(0)
            nbk = 0 if t < 4 else (1 if t < 10 else 0)
            gather_issue((t + 2) % 3, nbk, (t + 2) % B)

        pltpu.sync_copy(src.at[pl.ds(rbase, B)], is0)
        pltpu.sync_copy(dst.at[pl.ds(rbase, B)], id0)
        gather_issue(0, 0, 0)
        gather_issue(1, 0, 1)

        for t in range(2 * B):
            step(0, t, first=(t == 0))

        def body(m, carry):
            for t in range(2 * B):
                step(m, t, first=False)
            return carry

        lax.fori_loop(1, NB // 2, body, 0)
        gather_drain(0)
        gather_drain(1)
        scat_drain(2)
        plsc.subcore_barrier()

        for t in range(RPT // CC):
            r = s * RPT + t * CC
            pltpu.sync_copy(acc.at[pl.ds(r, CC)], gb0.at[pl.ds(0, CC)])
            pltpu.sync_copy(gb0.at[pl.ds(0, CC)], out.at[c, pl.ds(r, CC)])

    return sc_edge


_sc8 = _make_sc_edge(8)
_sc1 = _make_sc_edge(1)


def _pre1_body(x_ref, W_ref, As_ref, Ad_ref, ht_ref, dt_ref):
    h = jnp.dot(x_ref[...], W_ref[...], preferred_element_type=jnp.float32)
    sa = jnp.dot(h, As_ref[...], preferred_element_type=jnp.float32)
    ht_ref[...] = jnp.concatenate([h, sa], axis=1)
    dt_ref[...] = jnp.dot(h, Ad_ref[...], preferred_element_type=jnp.float32)


_pre1 = pl.pallas_call(
    _pre1_body,
    grid=(GRID,),
    in_specs=[
        pl.BlockSpec((BN, H), lambda i: (i, 0)),
        pl.BlockSpec((H, H), lambda i: (0, 0)),
        pl.BlockSpec((H, 16), lambda i: (0, 0)),
        pl.BlockSpec((H, 16), lambda i: (0, 0)),
    ],
    out_specs=[
        pl.BlockSpec((BN, CW), lambda i: (i, 0)),
        pl.BlockSpec((BN, 16), lambda i: (i, 0)),
    ],
    out_shape=[
        jax.ShapeDtypeStruct((N, CW), jnp.float32),
        jax.ShapeDtypeStruct((N, 16), jnp.float32),
    ],
)


def _epilogue(num, R_ref, b_ref, g_ref, be_ref):
    nsum = num[0] + num[1]
    den = jnp.dot(nsum[:, H:], R_ref[...],
                  preferred_element_type=jnp.float32) + 1e-16
    gat = nsum[:, :H] / den + b_ref[...]
    xe = jnp.where(gat > 0, gat, jnp.exp(gat) - 1.0)
    mu = jnp.mean(xe, axis=1, keepdims=True)
    var = jnp.mean((xe - mu) ** 2, axis=1, keepdims=True)
    return (xe - mu) / jnp.sqrt(var + 1e-5) * g_ref[...] + be_ref[...]


def _make_mid(has_res):
    def body(*refs):
        if has_res:
            (num_ref, res_ref, R_ref, b_ref, g_ref, be_ref,
             W_ref, As_ref, Ad_ref, x_ref, ht_ref, dt_ref) = refs
        else:
            (num_ref, R_ref, b_ref, g_ref, be_ref,
             W_ref, As_ref, Ad_ref, x_ref, ht_ref, dt_ref) = refs
        xn = _epilogue(num_ref[...], R_ref, b_ref, g_ref, be_ref)
        if has_res:
            xn = xn + res_ref[...]
        x_ref[...] = xn
        h = jnp.dot(xn, W_ref[...], preferred_element_type=jnp.float32)
        sa = jnp.dot(h, As_ref[...], preferred_element_type=jnp.float32)
        ht_ref[...] = jnp.concatenate([h, sa], axis=1)
        dt_ref[...] = jnp.dot(h, Ad_ref[...], preferred_element_type=jnp.float32)

    in_specs = [pl.BlockSpec((2, BN, CW), lambda i: (0, i, 0))]
    if has_res:
        in_specs.append(pl.BlockSpec((BN, H), lambda i: (i, 0)))
    in_specs += [
        pl.BlockSpec((16, H), lambda i: (0, 0)),
        pl.BlockSpec((1, H), lambda i: (0, 0)),
        pl.BlockSpec((1, H), lambda i: (0, 0)),
        pl.BlockSpec((1, H), lambda i: (0, 0)),
        pl.BlockSpec((H, H), lambda i: (0, 0)),
        pl.BlockSpec((H, 16), lambda i: (0, 0)),
        pl.BlockSpec((H, 16), lambda i: (0, 0)),
    ]
    return pl.pallas_call(
        body,
        grid=(GRID,),
        in_specs=in_specs,
        out_specs=[
            pl.BlockSpec((BN, H), lambda i: (i, 0)),
            pl.BlockSpec((BN, CW), lambda i: (i, 0)),
            pl.BlockSpec((BN, 16), lambda i: (i, 0)),
        ],
        out_shape=[
            jax.ShapeDtypeStruct((N, H), jnp.float32),
            jax.ShapeDtypeStruct((N, CW), jnp.float32),
            jax.ShapeDtypeStruct((N, 16), jnp.float32),
        ],
    )


_mid_nores = _make_mid(False)
_mid_res = _make_mid(True)


def _post_body(num_ref, res_ref, batch_ref, R_ref, b_ref, g_ref, be_ref,
               Wl1_ref, bl1_ref, Wl2_ref, bl2_ref, o_ref, acc, cnt):
    i = pl.program_id(0)
    h3 = _epilogue(num_ref[...], R_ref, b_ref, g_ref, be_ref) + res_ref[...]
    bvec = batch_ref[0, 0, :]
    onehot = (bvec[:, None] ==
              lax.broadcasted_iota(jnp.int32, (BN, G), 1)).astype(jnp.float32)
    dn = (((0,), (0,)), ((), ()))
    contrib = lax.dot_general(onehot, h3, dn, preferred_element_type=jnp.float32)
    ccontrib = lax.dot_general(onehot, jnp.ones((BN, H), jnp.float32), dn,
                               preferred_element_type=jnp.float32)

    @pl.when(i == 0)
    def _():
        acc[...] = contrib
        cnt[...] = ccontrib

    @pl.when(i > 0)
    def _():
        acc[...] = acc[...] + contrib
        cnt[...] = cnt[...] + ccontrib

    @pl.when(i == GRID - 1)
    def _():
        pooled = acc[...] / jnp.maximum(cnt[...], 1.0)
        t = jnp.dot(pooled, Wl1_ref[...],
                    preferred_element_type=jnp.float32) + bl1_ref[...]
        t = jnp.where(t > 0, t, jnp.exp(t) - 1.0)
        o_ref[...] = jnp.dot(t, Wl2_ref[...],
                             preferred_element_type=jnp.float32) + bl2_ref[...]


_post = pl.pallas_call(
    _post_body,
    grid=(GRID,),
    in_specs=[
        pl.BlockSpec((2, BN, CW), lambda i: (0, i, 0)),
        pl.BlockSpec((BN, H), lambda i: (i, 0)),
        pl.BlockSpec((1, 1, BN), lambda i: (i, 0, 0)),
        pl.BlockSpec((16, H), lambda i: (0, 0)),
        pl.BlockSpec((1, H), lambda i: (0, 0)),
        pl.BlockSpec((1, H), lambda i: (0, 0)),
        pl.BlockSpec((1, H), lambda i: (0, 0)),
        pl.BlockSpec((H, H), lambda i: (0, 0)),
        pl.BlockSpec((1, H), lambda i: (0, 0)),
        pl.BlockSpec((H, H), lambda i: (0, 0)),
        pl.BlockSpec((1, H), lambda i: (0, 0)),
    ],
    out_specs=pl.BlockSpec((G, H), lambda i: (0, 0)),
    out_shape=jax.ShapeDtypeStruct((G, H), jnp.float32),
    scratch_shapes=[
        pltpu.VMEM((G, H), jnp.float32),
        pltpu.VMEM((G, H), jnp.float32),
    ],
)


def kernel(x, edge_index, batch, W1, as1, ad1, b1, g1, be1, W2, as2, ad2, b2,
           g2, be2, W3, as3, ad3, b3, g3, be3, Wl1, bl1, Wl2, bl2):
    f32 = jnp.float32
    E = edge_index.shape[1]
    pad = IDX_ROWS * K - N - E
    loops = jnp.arange(N, dtype=jnp.int32)
    src = jnp.concatenate(
        [edge_index[0].astype(jnp.int32), loops,
         jnp.zeros((pad,), jnp.int32)]).reshape(IDX_ROWS, K)
    dst = jnp.concatenate(
        [edge_index[1].astype(jnp.int32), loops,
         jnp.full((pad,), N, jnp.int32)]).reshape(IDX_ROWS, K)

    eye8 = jnp.eye(8, dtype=f32)

    def head_proj(a):
        m = (eye8[:, None, :] * a[:, :, None]).reshape(H, 8)
        return jnp.pad(m, ((0, 0), (0, 8)))

    def one_proj(a):
        return jnp.pad(a.T, ((0, 0), (0, 15)))

    As1, Ad1 = head_proj(as1), head_proj(ad1)
    As2, Ad2 = one_proj(as2), one_proj(ad2)
    As3, Ad3 = one_proj(as3), one_proj(ad3)

    R8 = np.zeros((16, H), np.float32)
    for hh in range(8):
        R8[hh, 16 * hh:16 * hh + 16] = 1.0
    R8 = jnp.asarray(R8)
    R1 = np.zeros((16, H), np.float32)
    R1[0, :] = 1.0
    R1 = jnp.asarray(R1)

    rb = lambda v: v.reshape(1, H)
    batch3 = batch.astype(jnp.int32).reshape(GRID, 1, BN)

    ht1, dt1 = _pre1(x, W1, As1, Ad1)
    num1 = _sc8(ht1, dt1, src, dst)
    h1, ht2, dt2 = _mid_nores(num1, R8, rb(b1), rb(g1), rb(be1), W2, As2, Ad2)
    num2 = _sc1(ht2, dt2, src, dst)
    h2, ht3, dt3 = _mid_res(num2, h1, R1, rb(b2), rb(g2), rb(be2), W3, As3, Ad3)
    num3 = _sc1(ht3, dt3, src, dst)
    return _post(num3, h2, batch3, R1, rb(b3), rb(g3), rb(be3),
                 Wl1, rb(bl1), Wl2, rb(bl2))


# speedup vs baseline: 1.1076x; 1.0048x over previous
import functools

import jax
import jax.numpy as jnp
import numpy as np
from jax import lax
from jax.experimental import pallas as pl
from jax.experimental.pallas import tpu as pltpu
from jax.experimental.pallas import tpu_sc as plsc

N = 10000
H = 128
CW = 144
NR = 10240
K = 72
NW = 32
CH = 144
B = 6
NB = CH // B
EPW = K * CH
E_PAD = NW * EPW
IDX_ROWS = NW * CH + 8
G = 64
BN = 200
GRID = N // BN
RPT = NR // 16
CC = 64


def _make_sc_edge(heads):
    mesh = plsc.VectorSubcoreMesh(core_axis_name="c", subcore_axis_name="s")

    @functools.partial(
        pl.kernel,
        out_type=jax.ShapeDtypeStruct((2, NR, CW), jnp.float32),
        mesh=mesh,
        scratch_types=[
            pltpu.VMEM((K, CW), jnp.float32),
            pltpu.VMEM((K, CW), jnp.float32),
            pltpu.VMEM((K, CW), jnp.float32),
            pltpu.VMEM((K, 16), jnp.float32),
            pltpu.VMEM((K, 16), jnp.float32),
            pltpu.VMEM((K, 16), jnp.float32),
            pltpu.VMEM((B, K), jnp.int32),
            pltpu.VMEM((B, K), jnp.int32),
            pltpu.VMEM((B, K), jnp.int32),
            pltpu.VMEM((B, K), jnp.int32),
            pltpu.VMEM_SHARED((NR, CW), jnp.float32),
        ] + [pltpu.SemaphoreType.DMA] * 13,
        compiler_params=pltpu.CompilerParams(use_tc_tiling_on_sc=False),
    )
    def sc_edge(ht, dt, src, dst, out, gb0, gb1, gb2, db0, db1, db2,
                is0, is1, id0, id1, acc,
                g0h, g1h, g2h, g0d, g1d, g2d, s0, s1, s2, b0s, b1s, b0d, b1d):
        c = lax.axis_index("c")
        s = lax.axis_index("s")
        w = s * 2 + c
        rbase = w * CH

        gbufs, dbufs = (gb0, gb1, gb2), (db0, db1, db2)
        isrc, idst = (is0, is1), (id0, id1)
        ghs, gds = (g0h, g1h, g2h), (g0d, g1d, g2d)
        scs = (s0, s1, s2)
        ibs, ibd = (b0s, b1s), (b0d, b1d)

        zero = jnp.zeros((16,), jnp.float32)

        def zrow(i, carry):
            for k in range(CW // 16):
                gb0[i, pl.ds(16 * k, 16)] = zero
            return carry

        lax.fori_loop(0, CC, zrow, 0)
        for t in range(RPT // CC):
            pltpu.async_copy(gb0.at[pl.ds(0, CC)],
                             acc.at[pl.ds(s * RPT + t * CC, CC)], s0)
        for t in range(RPT // CC):
            pltpu.make_async_copy(gb0.at[pl.ds(0, CC)],
                                  acc.at[pl.ds(s * RPT, CC)], s0).wait()
        plsc.subcore_barrier()

        def blk_issue(bi, n):
            r = rbase + n * B
            pltpu.async_copy(src.at[pl.ds(r, B)], isrc[bi], ibs[bi])
            pltpu.async_copy(dst.at[pl.ds(r, B)], idst[bi], ibd[bi])

        def blk_drain(bi):
            pltpu.make_async_copy(src.at[pl.ds(0, B)], isrc[bi], ibs[bi]).wait()
            pltpu.make_async_copy(dst.at[pl.ds(0, B)], idst[bi], ibd[bi]).wait()

        def gather_issue(gi, bi, j):
            pltpu.async_copy(ht.at[isrc[bi].at[j]], gbufs[gi], ghs[gi])
            pltpu.async_copy(dt.at[idst[bi].at[j]], dbufs[gi], gds[gi])

        def gather_drain(gi):
            pltpu.make_async_copy(ht.at[isrc[0].at[0]], gbufs[gi],
                                  ghs[gi]).wait()
            pltpu.make_async_copy(dt.at[idst[0].at[0]], dbufs[gi],
                                  gds[gi]).wait()

        def compute(gi):
            gb, db = gbufs[gi], dbufs[gi]

            def edge(i2, carry2):
                for u in range(2):
                    i = i2 * 2 + u
                    lg = gb[i, pl.ds(H, 16)] + db[i, :]
                    p = jnp.exp(jnp.where(lg >= 0.0, lg, lg * 0.2))
                    gb[i, pl.ds(H, 16)] = p
                    if heads == 1:
                        p0 = p.at[jnp.zeros((16,), jnp.int32)].get(
                            mode="promise_in_bounds")
                    for j8 in range(8):
                        pj = (p.at[jnp.full((16,), j8, jnp.int32)].get(
                                  mode="promise_in_bounds")
                              if heads == 8 else p0)
                        gb[i, pl.ds(16 * j8, 16)] = (
                            gb[i, pl.ds(16 * j8, 16)] * pj)
                return carry2

            lax.fori_loop(0, K // 2, edge, 0)

        def scat_issue(gi, bi, j):
            pltpu.async_copy(gbufs[gi], acc.at[idst[bi].at[j]], scs[gi],
                             add=True)

        def scat_drain(gi):
            pltpu.make_async_copy(gbufs[gi], acc.at[idst[0].at[0]],
                                  scs[gi]).wait()

        def step(m, t, first):
            gi = t % 3
            gather_drain(gi)
            compute(gi)
            scat_issue(gi, t // B, t % B)
            if not first:
                scat_drain((t + 2) % 3)
            if t == 0:
                blk_issue(1, 2 * m + 1)
            if t == 4:
                blk_drain(1)
            if t == 6:
                blk_issue(0, 2 * m + 2)
            if t == 10:
                blk_drain(0)
            nbk = 0 if t < 4 else (1 if t < 10 else 0)
            gather_issue((t + 2) % 3, nbk, (t + 2) % B)

        pltpu.sync_copy(src.at[pl.ds(rbase, B)], is0)
        pltpu.sync_copy(dst.at[pl.ds(rbase, B)], id0)
        gather_issue(0, 0, 0)
        gather_issue(1, 0, 1)

        for t in range(2 * B):
            step(0, t, first=(t == 0))

        def body(m, carry):
            for t in range(2 * B):
                step(m, t, first=False)
            return carry

        lax.fori_loop(1, NB // 2, body, 0)
        gather_drain(0)
        gather_drain(1)
        scat_drain(2)
        plsc.subcore_barrier()

        for t in range(RPT // CC):
            r = s * RPT + t * CC
            pltpu.async_copy(acc.at[pl.ds(r, CC)], out.at[c, pl.ds(r, CC)], s1)
        for t in range(RPT // CC):
            pltpu.make_async_copy(acc.at[pl.ds(s * RPT, CC)],
                                  out.at[c, pl.ds(s * RPT, CC)], s1).wait()

    return sc_edge


_sc8 = _make_sc_edge(8)
_sc1 = _make_sc_edge(1)


def _pre1_body(x_ref, W_ref, As_ref, Ad_ref, ht_ref, dt_ref):
    h = jnp.dot(x_ref[...], W_ref[...], preferred_element_type=jnp.float32)
    sa = jnp.dot(h, As_ref[...], preferred_element_type=jnp.float32)
    ht_ref[...] = jnp.concatenate([h, sa], axis=1)
    dt_ref[...] = jnp.dot(h, Ad_ref[...], preferred_element_type=jnp.float32)


_pre1 = pl.pallas_call(
    _pre1_body,
    grid=(GRID,),
    in_specs=[
        pl.BlockSpec((BN, H), lambda i: (i, 0)),
        pl.BlockSpec((H, H), lambda i: (0, 0)),
        pl.BlockSpec((H, 16), lambda i: (0, 0)),
        pl.BlockSpec((H, 16), lambda i: (0, 0)),
    ],
    out_specs=[
        pl.BlockSpec((BN, CW), lambda i: (i, 0)),
        pl.BlockSpec((BN, 16), lambda i: (i, 0)),
    ],
    out_shape=[
        jax.ShapeDtypeStruct((N, CW), jnp.float32),
        jax.ShapeDtypeStruct((N, 16), jnp.float32),
    ],
)


def _epilogue(num, R_ref, b_ref, g_ref, be_ref):
    nsum = num[0] + num[1]
    den = jnp.dot(nsum[:, H:], R_ref[...],
                  preferred_element_type=jnp.float32) + 1e-16
    gat = nsum[:, :H] / den + b_ref[...]
    xe = jnp.where(gat > 0, gat, jnp.exp(gat) - 1.0)
    mu = jnp.mean(xe, axis=1, keepdims=True)
    var = jnp.mean((xe - mu) ** 2, axis=1, keepdims=True)
    return (xe - mu) / jnp.sqrt(var + 1e-5) * g_ref[...] + be_ref[...]


def _make_mid(has_res):
    def body(*refs):
        if has_res:
            (num_ref, res_ref, R_ref, b_ref, g_ref, be_ref,
             W_ref, As_ref, Ad_ref, x_ref, ht_ref, dt_ref) = refs
        else:
            (num_ref, R_ref, b_ref, g_ref, be_ref,
             W_ref, As_ref, Ad_ref, x_ref, ht_ref, dt_ref) = refs
        xn = _epilogue(num_ref[...], R_ref, b_ref, g_ref, be_ref)
        if has_res:
            xn = xn + res_ref[...]
        x_ref[...] = xn
        h = jnp.dot(xn, W_ref[...], preferred_element_type=jnp.float32)
        sa = jnp.dot(h, As_ref[...], preferred_element_type=jnp.float32)
        ht_ref[...] = jnp.concatenate([h, sa], axis=1)
        dt_ref[...] = jnp.dot(h, Ad_ref[...], preferred_element_type=jnp.float32)

    in_specs = [pl.BlockSpec((2, BN, CW), lambda i: (0, i, 0))]
    if has_res:
        in_specs.append(pl.BlockSpec((BN, H), lambda i: (i, 0)))
    in_specs += [
        pl.BlockSpec((16, H), lambda i: (0, 0)),
        pl.BlockSpec((1, H), lambda i: (0, 0)),
        pl.BlockSpec((1, H), lambda i: (0, 0)),
        pl.BlockSpec((1, H), lambda i: (0, 0)),
        pl.BlockSpec((H, H), lambda i: (0, 0)),
        pl.BlockSpec((H, 16), lambda i: (0, 0)),
        pl.BlockSpec((H, 16), lambda i: (0, 0)),
    ]
    return pl.pallas_call(
        body,
        grid=(GRID,),
        in_specs=in_specs,
        out_specs=[
            pl.BlockSpec((BN, H), lambda i: (i, 0)),
            pl.BlockSpec((BN, CW), lambda i: (i, 0)),
            pl.BlockSpec((BN, 16), lambda i: (i, 0)),
        ],
        out_shape=[
            jax.ShapeDtypeStruct((N, H), jnp.float32),
            jax.ShapeDtypeStruct((N, CW), jnp.float32),
            jax.ShapeDtypeStruct((N, 16), jnp.float32),
        ],
    )


_mid_nores = _make_mid(False)
_mid_res = _make_mid(True)


def _post_body(num_ref, res_ref, batch_ref, R_ref, b_ref, g_ref, be_ref,
               Wl1_ref, bl1_ref, Wl2_ref, bl2_ref, o_ref, acc, cnt):
    i = pl.program_id(0)
    h3 = _epilogue(num_ref[...], R_ref, b_ref, g_ref, be_ref) + res_ref[...]
    bvec = batch_ref[0, 0, :]
    onehot = (bvec[:, None] ==
              lax.broadcasted_iota(jnp.int32, (BN, G), 1)).astype(jnp.float32)
    dn = (((0,), (0,)), ((), ()))
    contrib = lax.dot_general(onehot, h3, dn, preferred_element_type=jnp.float32)
    ccontrib = lax.dot_general(onehot, jnp.ones((BN, H), jnp.float32), dn,
                               preferred_element_type=jnp.float32)

    @pl.when(i == 0)
    def _():
        acc[...] = contrib
        cnt[...] = ccontrib

    @pl.when(i > 0)
    def _():
        acc[...] = acc[...] + contrib
        cnt[...] = cnt[...] + ccontrib

    @pl.when(i == GRID - 1)
    def _():
        pooled = acc[...] / jnp.maximum(cnt[...], 1.0)
        t = jnp.dot(pooled, Wl1_ref[...],
                    preferred_element_type=jnp.float32) + bl1_ref[...]
        t = jnp.where(t > 0, t, jnp.exp(t) - 1.0)
        o_ref[...] = jnp.dot(t, Wl2_ref[...],
                             preferred_element_type=jnp.float32) + bl2_ref[...]


_post = pl.pallas_call(
    _post_body,
    grid=(GRID,),
    in_specs=[
        pl.BlockSpec((2, BN, CW), lambda i: (0, i, 0)),
        pl.BlockSpec((BN, H), lambda i: (i, 0)),
        pl.BlockSpec((1, 1, BN), lambda i: (i, 0, 0)),
        pl.BlockSpec((16, H), lambda i: (0, 0)),
        pl.BlockSpec((1, H), lambda i: (0, 0)),
        pl.BlockSpec((1, H), lambda i: (0, 0)),
        pl.BlockSpec((1, H), lambda i: (0, 0)),
        pl.BlockSpec((H, H), lambda i: (0, 0)),
        pl.BlockSpec((1, H), lambda i: (0, 0)),
        pl.BlockSpec((H, H), lambda i: (0, 0)),
        pl.BlockSpec((1, H), lambda i: (0, 0)),
    ],
    out_specs=pl.BlockSpec((G, H), lambda i: (0, 0)),
    out_shape=jax.ShapeDtypeStruct((G, H), jnp.float32),
    scratch_shapes=[
        pltpu.VMEM((G, H), jnp.float32),
        pltpu.VMEM((G, H), jnp.float32),
    ],
)


def kernel(x, edge_index, batch, W1, as1, ad1, b1, g1, be1, W2, as2, ad2, b2,
           g2, be2, W3, as3, ad3, b3, g3, be3, Wl1, bl1, Wl2, bl2):
    f32 = jnp.float32
    E = edge_index.shape[1]
    pad = IDX_ROWS * K - N - E
    loops = jnp.arange(N, dtype=jnp.int32)
    src = jnp.concatenate(
        [edge_index[0].astype(jnp.int32), loops,
         jnp.zeros((pad,), jnp.int32)]).reshape(IDX_ROWS, K)
    dst = jnp.concatenate(
        [edge_index[1].astype(jnp.int32), loops,
         jnp.full((pad,), N, jnp.int32)]).reshape(IDX_ROWS, K)

    eye8 = jnp.eye(8, dtype=f32)

    def head_proj(a):
        m = (eye8[:, None, :] * a[:, :, None]).reshape(H, 8)
        return jnp.pad(m, ((0, 0), (0, 8)))

    def one_proj(a):
        return jnp.pad(a.T, ((0, 0), (0, 15)))

    As1, Ad1 = head_proj(as1), head_proj(ad1)
    As2, Ad2 = one_proj(as2), one_proj(ad2)
    As3, Ad3 = one_proj(as3), one_proj(ad3)

    R8 = np.zeros((16, H), np.float32)
    for hh in range(8):
        R8[hh, 16 * hh:16 * hh + 16] = 1.0
    R8 = jnp.asarray(R8)
    R1 = np.zeros((16, H), np.float32)
    R1[0, :] = 1.0
    R1 = jnp.asarray(R1)

    rb = lambda v: v.reshape(1, H)
    batch3 = batch.astype(jnp.int32).reshape(GRID, 1, BN)

    ht1, dt1 = _pre1(x, W1, As1, Ad1)
    num1 = _sc8(ht1, dt1, src, dst)
    h1, ht2, dt2 = _mid_nores(num1, R8, rb(b1), rb(g1), rb(be1), W2, As2, Ad2)
    num2 = _sc1(ht2, dt2, src, dst)
    h2, ht3, dt3 = _mid_res(num2, h1, R1, rb(b2), rb(g2), rb(be2), W3, As3, Ad3)
    num3 = _sc1(ht3, dt3, src, dst)
    return _post(num3, h2, batch3, R1, rb(b3), rb(g3), rb(be3),
                 Wl1, rb(bl1), Wl2, rb(bl2))
